# baseline (speedup 1.0000x reference)
import functools

import jax
import jax.numpy as jnp
from jax import lax
from jax.experimental import pallas as pl
from jax.experimental.pallas import tpu as pltpu
from jax.experimental.pallas import tpu_sc as plsc

N = 10000
E = 320000
OUT_C = 64
HALF = 32
K = 10
ALPHA = 0.1
BN_EPS = 1e-5

NC = 2
NS = 16
NP = 10112
SEG = NP // NS
CH = 128

EPT = E // NS
C2 = 400
NCH2 = EPT // C2
NCHP = 51
SB = 158
NSB = SEG // SB

EPW = E // (NC * NS)
DC = 400
DNC = EPW // DC


def _mlp_prep_body(x_ref, w0_ref, b0_ref, gamma_ref, beta_ref, w1_ref, b1_ref,
                   deg2_ref, u0_ref, g_ref, coef_ref):
    x = x_ref[...]
    h = jnp.dot(x, w0_ref[...], preferred_element_type=jnp.float32)
    h = h + b0_ref[...][None, :]
    mu = jnp.mean(h, axis=0, keepdims=True)
    d = h - mu
    var = jnp.mean(d * d, axis=0, keepdims=True)
    h = d / jnp.sqrt(var + BN_EPS) * gamma_ref[...][None, :] + beta_ref[...][None, :]
    h = jnp.maximum(h, 0.0)
    out0 = (
        jnp.dot(h, w1_ref[...], preferred_element_type=jnp.float32)
        + b1_ref[...][None, :]
    )
    deg = deg2_ref[0] + deg2_ref[1]
    sinv = 1.0 / jnp.sqrt(deg)
    coef_ref[...] = (1.0 - ALPHA) * sinv * sinv
    s_n = sinv[:N, 0:1]
    u0 = out0 * s_n
    g = ALPHA * u0
    zpad = jnp.zeros((NP - N, HALF), jnp.float32)
    u0_ref[pl.ds(0, N), :] = u0[:, :HALF]
    u0_ref[pl.ds(N, NP - N), :] = zpad
    u0_ref[pl.ds(NP, N), :] = u0[:, HALF:]
    u0_ref[pl.ds(NP + N, NP - N), :] = zpad
    g_ref[pl.ds(0, N), :] = g[:, :HALF]
    g_ref[pl.ds(N, NP - N), :] = zpad
    g_ref[pl.ds(NP, N), :] = g[:, HALF:]
    g_ref[pl.ds(NP + N, NP - N), :] = zpad


_mlp_prep = pl.pallas_call(
    _mlp_prep_body,
    out_shape=[
        jax.ShapeDtypeStruct((2 * NP, HALF), jnp.float32),
        jax.ShapeDtypeStruct((2 * NP, HALF), jnp.float32),
        jax.ShapeDtypeStruct((NP, 16), jnp.float32),
    ],
)


def _deg_body(edge_ref, deg_out, idx0, idx1, ones, seg, degsp, sd0, sd1):
    c = lax.axis_index("c")
    s = lax.axis_index("s")
    w = s * NC + c
    ebase = w * EPW

    def fill_ones(i, carry):
        ones[i] = jnp.zeros((16,), jnp.float32) + 1.0
        return carry

    lax.fori_loop(0, DC, fill_ones, 0)

    def fill_seg0(i, carry):
        seg[i] = jnp.zeros((16,), jnp.float32)
        return carry

    lax.fori_loop(0, SEG, fill_seg0, 0)

    @pl.when(c == 0)
    def _fill_seg1():
        def fill_seg1(i, carry):
            seg[i] = jnp.zeros((16,), jnp.float32) + 1.0
            return carry

        lax.fori_loop(0, SEG, fill_seg1, 0)
    pltpu.sync_copy(seg, degsp.at[pl.ds(s * SEG, SEG)])
    plsc.subcore_barrier()

    idxs = (idx0, idx1)
    dsems = (sd0, sd1)

    def stage_issue(j, b):
        pltpu.async_copy(edge_ref.at[pl.ds(E + ebase + j * DC, DC)],
                         idxs[b].at[0], dsems[b])

    def stage_wait(b):
        pltpu.make_async_copy(edge_ref.at[pl.ds(0, DC)], idxs[b].at[0],
                              dsems[b]).wait()

    def dscat(b):
        pltpu.sync_copy(ones, degsp.at[idxs[b].at[0]], add=True)

    stage_issue(0, 0)

    def dpair(gp, carry):
        j0 = 2 * gp
        j1 = 2 * gp + 1
        stage_wait(0)
        stage_issue(j1, 1)
        dscat(0)
        stage_wait(1)

        @pl.when(j1 + 1 < DNC)
        def _():
            stage_issue(j1 + 1, 0)

        dscat(1)
        return carry

    lax.fori_loop(0, DNC // 2, dpair, 0)
    stage_wait(0)
    dscat(0)

    plsc.subcore_barrier()
    pltpu.sync_copy(degsp.at[pl.ds(s * SEG, SEG)], seg)
    pltpu.sync_copy(seg, deg_out.at[c, pl.ds(s * SEG, SEG)])


_deg_kernel = functools.partial(
    pl.kernel,
    out_type=jax.ShapeDtypeStruct((NC, NP, 16), jnp.float32),
    mesh=plsc.VectorSubcoreMesh(core_axis_name="c", subcore_axis_name="s"),
    compiler_params=pltpu.CompilerParams(use_tc_tiling_on_sc=False),
    scratch_types=[
        pltpu.VMEM((1, DC), jnp.int32),
        pltpu.VMEM((1, DC), jnp.int32),
        pltpu.VMEM((DC, 16), jnp.float32),
        pltpu.VMEM((SEG, 16), jnp.float32),
        pltpu.VMEM_SHARED((NP + 16, 16), jnp.float32),
        pltpu.SemaphoreType.DMA,
        pltpu.SemaphoreType.DMA,
    ],
)(_deg_body)


def _appnp_body(edge_ref, u0_ref, coef_ref, g_ref, u_ref,
                src_idx, dst_idx, gbuf0, gbuf1, gbuf2, bbuf, gvec,
                cvec, agg, sg0, sg1, sg2, ssc0, ssc1, ssc2):
    c = lax.axis_index("c")
    t = lax.axis_index("s")
    ebase = t * EPT
    rbase = t * SEG
    cnp = c * NP

    def stage_row(j, carry):
        pltpu.sync_copy(edge_ref.at[pl.ds(ebase + j * C2, C2)], src_idx.at[j])
        pltpu.sync_copy(edge_ref.at[pl.ds(E + ebase + j * C2, C2)], dst_idx.at[j])
        return carry

    lax.fori_loop(0, NCH2, stage_row, 0)

    def fill_pad(l, carry):
        src_idx[NCH2, pl.ds(l * 16, 16)] = jnp.zeros((16,), jnp.int32)
        dst_idx[NCH2, pl.ds(l * 16, 16)] = jnp.zeros((16,), jnp.int32) + N
        return carry

    lax.fori_loop(0, C2 // 16, fill_pad, 0)

    def add_off(j, carry):
        def add_lane(l, inner):
            v = src_idx[j, pl.ds(l * 16, 16)]
            src_idx[j, pl.ds(l * 16, 16)] = v + cnp
            return inner

        return lax.fori_loop(0, C2 // 16, add_lane, carry)

    lax.fori_loop(0, NCHP, add_off, 0)

    def init_sub(sb, carry):
        off = rbase + sb * SB
        pltpu.sync_copy(u0_ref.at[pl.ds(cnp + off, SB)], bbuf)
        pltpu.sync_copy(bbuf, u_ref.at[pl.ds(cnp + off, SB)])
        pltpu.sync_copy(bbuf, agg.at[pl.ds(off, SB)])
        return carry

    lax.fori_loop(0, NSB, init_sub, 0)
    plsc.subcore_barrier()

    bufs = (gbuf0, gbuf1, gbuf2)
    gsems = (sg0, sg1, sg2)
    ssems = (ssc0, ssc1, ssc2)

    def gather_issue(j, b):
        pltpu.async_copy(u_ref.at[src_idx.at[j]], bufs[b], gsems[b])

    def gather_wait(b):
        pltpu.make_async_copy(u_ref.at[pl.ds(0, C2)], bufs[b], gsems[b]).wait()

    def scatter_issue(j, b):
        pltpu.async_copy(bufs[b], agg.at[dst_idx.at[j]], ssems[b], add=True)

    def scatter_wait(b):
        pltpu.make_async_copy(bufs[b], agg.at[pl.ds(0, C2)], ssems[b]).wait()

    def round_body(k, carry):
        gather_issue(0, 0)

        def group(g, c2):
            for b in range(3):
                j = 3 * g + b
                bn = (b + 1) % 3
                gather_wait(b)

                @pl.when(j >= 2)
                def _():
                    scatter_wait(bn)

                @pl.when(j + 1 < NCHP)
                def _():
                    gather_issue(j + 1, bn)

                scatter_issue(j, b)
            return c2

        lax.fori_loop(0, NCHP // 3, group, 0)
        scatter_wait(1)
        scatter_wait(2)
        plsc.subcore_barrier()

        def blend_sub(sb, carry2):
            off = rbase + sb * SB
            pltpu.sync_copy(coef_ref.at[pl.ds(off, SB)], cvec)
            pltpu.sync_copy(g_ref.at[pl.ds(cnp + off, SB)], gvec)
            pltpu.sync_copy(agg.at[pl.ds(off, SB)], bbuf)

            def blend_row(i, c3):
                ci = cvec[i][0]
                bbuf[i, pl.ds(0, 16)] = (
                    bbuf[i, pl.ds(0, 16)] * ci + gvec[i, pl.ds(0, 16)]
                )
                bbuf[i, pl.ds(16, 16)] = (
                    bbuf[i, pl.ds(16, 16)] * ci + gvec[i, pl.ds(16, 16)]
                )
                return c3

            lax.fori_loop(0, SB, blend_row, 0)
            pltpu.sync_copy(bbuf, u_ref.at[pl.ds(cnp + off, SB)])
            pltpu.sync_copy(bbuf, agg.at[pl.ds(off, SB)])
            return carry2

        lax.fori_loop(0, NSB, blend_sub, 0)
        plsc.subcore_barrier()
        return carry

    lax.fori_loop(0, K, round_body, 0)


_appnp = functools.partial(
    pl.kernel,
    out_type=jax.ShapeDtypeStruct((2 * NP, HALF), jnp.float32),
    mesh=plsc.VectorSubcoreMesh(core_axis_name="c", subcore_axis_name="s"),
    compiler_params=pltpu.CompilerParams(use_tc_tiling_on_sc=False),
    scratch_types=[
        pltpu.VMEM((NCHP, C2), jnp.int32),
        pltpu.VMEM((NCHP, C2), jnp.int32),
        pltpu.VMEM((C2, HALF), jnp.float32),
        pltpu.VMEM((C2, HALF), jnp.float32),
        pltpu.VMEM((C2, HALF), jnp.float32),
        pltpu.VMEM((SB, HALF), jnp.float32),
        pltpu.VMEM((SB, HALF), jnp.float32),
        pltpu.VMEM((SB, 16), jnp.float32),
        pltpu.VMEM_SHARED((NP + 16, HALF), jnp.float32),
        pltpu.SemaphoreType.DMA,
        pltpu.SemaphoreType.DMA,
        pltpu.SemaphoreType.DMA,
        pltpu.SemaphoreType.DMA,
        pltpu.SemaphoreType.DMA,
        pltpu.SemaphoreType.DMA,
    ],
)(_appnp_body)


def _final_body(u_ref, deg2_ref, out_ref):
    deg = deg2_ref[0] + deg2_ref[1]
    srt = jnp.sqrt(deg[:N, 0:1])
    u64 = jnp.concatenate([u_ref[pl.ds(0, N), :], u_ref[pl.ds(NP, N), :]], axis=1)
    o = u64 * srt
    m = jnp.max(o, axis=1, keepdims=True)
    e = o - m
    lse = jnp.log(jnp.sum(jnp.exp(e), axis=1, keepdims=True))
    out_ref[...] = e - lse


_final = pl.pallas_call(
    _final_body,
    out_shape=jax.ShapeDtypeStruct((N, OUT_C), jnp.float32),
)


def kernel(x, edge_index, W0, b0, gamma, beta, W1, b1):
    edge_flat = edge_index.reshape(2 * E)
    deg2 = _deg_kernel(edge_flat)
    u0, g, coef = _mlp_prep(x, W0, b0, gamma, beta, W1, b1, deg2)
    u = _appnp(edge_flat, u0, coef, g)
    return _final(u, deg2)


# speedup vs baseline: 1.7122x; 1.7122x over previous
import functools

import jax
import jax.numpy as jnp
from jax import lax
from jax.experimental import pallas as pl
from jax.experimental.pallas import tpu as pltpu
from jax.experimental.pallas import tpu_sc as plsc

N = 10000
E = 320000
OUT_C = 64
HALF = 32
K = 10
ALPHA = 0.1
BN_EPS = 1e-5

NC = 2
NS = 16
NP = 10112
SEG = NP // NS
CH = 128

EPT = E // NS
C2 = 400
NCH2 = EPT // C2
SB = 158
NSB = SEG // SB

EPW = E // (NC * NS)
DC = 400
DNC = EPW // DC


def _mlp_prep_body(x_ref, w0_ref, b0_ref, gamma_ref, beta_ref, w1_ref, b1_ref,
                   deg2_ref, u0_ref, g_ref, coef_ref):
    x = x_ref[...]
    h = jnp.dot(x, w0_ref[...], preferred_element_type=jnp.float32)
    h = h + b0_ref[...][None, :]
    mu = jnp.mean(h, axis=0, keepdims=True)
    d = h - mu
    var = jnp.mean(d * d, axis=0, keepdims=True)
    h = d / jnp.sqrt(var + BN_EPS) * gamma_ref[...][None, :] + beta_ref[...][None, :]
    h = jnp.maximum(h, 0.0)
    out0 = (
        jnp.dot(h, w1_ref[...], preferred_element_type=jnp.float32)
        + b1_ref[...][None, :]
    )
    deg = deg2_ref[0] + deg2_ref[1]
    sinv = 1.0 / jnp.sqrt(deg)
    coef_ref[...] = (1.0 - ALPHA) * sinv * sinv
    s_n = sinv[:N, 0:1]
    u0 = out0 * s_n
    g = ALPHA * u0
    zpad = jnp.zeros((NP - N, HALF), jnp.float32)
    u0_ref[pl.ds(0, N), :] = u0[:, :HALF]
    u0_ref[pl.ds(N, NP - N), :] = zpad
    u0_ref[pl.ds(NP, N), :] = u0[:, HALF:]
    u0_ref[pl.ds(NP + N, NP - N), :] = zpad
    g_ref[pl.ds(0, N), :] = g[:, :HALF]
    g_ref[pl.ds(N, NP - N), :] = zpad
    g_ref[pl.ds(NP, N), :] = g[:, HALF:]
    g_ref[pl.ds(NP + N, NP - N), :] = zpad


_mlp_prep = pl.pallas_call(
    _mlp_prep_body,
    out_shape=[
        jax.ShapeDtypeStruct((2 * NP, HALF), jnp.float32),
        jax.ShapeDtypeStruct((2 * NP, HALF), jnp.float32),
        jax.ShapeDtypeStruct((NP, 16), jnp.float32),
    ],
)


def _deg_body(edge_ref, deg_out, idx0, idx1, ones, seg, degsp, sd0, sd1):
    c = lax.axis_index("c")
    s = lax.axis_index("s")
    w = s * NC + c
    ebase = w * EPW

    def fill_ones(i, carry):
        ones[i] = jnp.zeros((16,), jnp.float32) + 1.0
        return carry

    lax.fori_loop(0, DC, fill_ones, 0)

    def fill_seg0(i, carry):
        seg[i] = jnp.zeros((16,), jnp.float32)
        return carry

    lax.fori_loop(0, SEG, fill_seg0, 0)

    @pl.when(c == 0)
    def _fill_seg1():
        def fill_seg1(i, carry):
            seg[i] = jnp.zeros((16,), jnp.float32) + 1.0
            return carry

        lax.fori_loop(0, SEG, fill_seg1, 0)
    pltpu.sync_copy(seg, degsp.at[pl.ds(s * SEG, SEG)])
    plsc.subcore_barrier()

    idxs = (idx0, idx1)
    dsems = (sd0, sd1)

    def stage_issue(j, b):
        pltpu.async_copy(edge_ref.at[pl.ds(E + ebase + j * DC, DC)],
                         idxs[b].at[0], dsems[b])

    def stage_wait(b):
        pltpu.make_async_copy(edge_ref.at[pl.ds(0, DC)], idxs[b].at[0],
                              dsems[b]).wait()

    def dscat(b):
        pltpu.sync_copy(ones, degsp.at[idxs[b].at[0]], add=True)

    stage_issue(0, 0)

    def dpair(gp, carry):
        j0 = 2 * gp
        j1 = 2 * gp + 1
        stage_wait(0)
        stage_issue(j1, 1)
        dscat(0)
        stage_wait(1)

        @pl.when(j1 + 1 < DNC)
        def _():
            stage_issue(j1 + 1, 0)

        dscat(1)
        return carry

    lax.fori_loop(0, DNC // 2, dpair, 0)
    stage_wait(0)
    dscat(0)

    plsc.subcore_barrier()
    pltpu.sync_copy(degsp.at[pl.ds(s * SEG, SEG)], seg)
    pltpu.sync_copy(seg, deg_out.at[c, pl.ds(s * SEG, SEG)])


_deg_kernel = functools.partial(
    pl.kernel,
    out_type=jax.ShapeDtypeStruct((NC, NP, 16), jnp.float32),
    mesh=plsc.VectorSubcoreMesh(core_axis_name="c", subcore_axis_name="s"),
    compiler_params=pltpu.CompilerParams(use_tc_tiling_on_sc=False),
    scratch_types=[
        pltpu.VMEM((1, DC), jnp.int32),
        pltpu.VMEM((1, DC), jnp.int32),
        pltpu.VMEM((DC, 16), jnp.float32),
        pltpu.VMEM((SEG, 16), jnp.float32),
        pltpu.VMEM_SHARED((NP + 16, 16), jnp.float32),
        pltpu.SemaphoreType.DMA,
        pltpu.SemaphoreType.DMA,
    ],
)(_deg_body)


def _appnp_body(edge_ref, u0_ref, coef_ref, g_ref, u_ref,
                src_idx, dst_idx, gbuf0, gbuf1, bbuf, gvec,
                cvec, agg, sg0, sg1):
    c = lax.axis_index("c")
    t = lax.axis_index("s")
    ebase = t * EPT
    rbase = t * SEG
    cnp = c * NP

    def stage_row(j, carry):
        pltpu.sync_copy(edge_ref.at[pl.ds(ebase + j * C2, C2)], src_idx.at[j])
        pltpu.sync_copy(edge_ref.at[pl.ds(E + ebase + j * C2, C2)], dst_idx.at[j])
        return carry

    lax.fori_loop(0, NCH2, stage_row, 0)

    def add_off(j, carry):
        def add_lane(l, inner):
            v = src_idx[j, pl.ds(l * 16, 16)]
            src_idx[j, pl.ds(l * 16, 16)] = v + cnp
            return inner

        return lax.fori_loop(0, C2 // 16, add_lane, carry)

    lax.fori_loop(0, NCH2, add_off, 0)

    pltpu.sync_copy(coef_ref.at[pl.ds(rbase, SEG)], cvec)
    pltpu.sync_copy(g_ref.at[pl.ds(cnp + rbase, SEG)], gvec)

    def init_sub(sb, carry):
        off = rbase + sb * SB
        pltpu.sync_copy(u0_ref.at[pl.ds(cnp + off, SB)], bbuf)
        pltpu.sync_copy(bbuf, u_ref.at[pl.ds(cnp + off, SB)])
        pltpu.sync_copy(bbuf, agg.at[pl.ds(off, SB)])
        return carry

    lax.fori_loop(0, NSB, init_sub, 0)
    plsc.subcore_barrier()

    bufs = (gbuf0, gbuf1)
    sems = (sg0, sg1)

    def gather_issue(j, b):
        pltpu.async_copy(u_ref.at[src_idx.at[j]], bufs[b], sems[b])

    def gather_wait(b):
        pltpu.make_async_copy(u_ref.at[pl.ds(0, C2)], bufs[b], sems[b]).wait()

    def round_body(k, carry):
        gather_issue(0, 0)

        def pair(gp, c2):
            j0 = 2 * gp
            j1 = 2 * gp + 1
            gather_wait(0)
            gather_issue(j1, 1)
            pltpu.sync_copy(bufs[0], agg.at[dst_idx.at[j0]], add=True)
            gather_wait(1)

            @pl.when(j1 + 1 < NCH2)
            def _():
                gather_issue(j1 + 1, 0)

            pltpu.sync_copy(bufs[1], agg.at[dst_idx.at[j1]], add=True)
            return c2

        lax.fori_loop(0, NCH2 // 2, pair, 0)
        plsc.subcore_barrier()

        def blend_sub(sb, carry2):
            off = rbase + sb * SB
            pltpu.sync_copy(agg.at[pl.ds(off, SB)], bbuf)

            def blend_row(i, c3):
                ci = cvec[sb * SB + i][0]
                bbuf[i, pl.ds(0, 16)] = (
                    bbuf[i, pl.ds(0, 16)] * ci + gvec[sb * SB + i, pl.ds(0, 16)]
                )
                bbuf[i, pl.ds(16, 16)] = (
                    bbuf[i, pl.ds(16, 16)] * ci + gvec[sb * SB + i, pl.ds(16, 16)]
                )
                return c3

            lax.fori_loop(0, SB, blend_row, 0)
            pltpu.sync_copy(bbuf, u_ref.at[pl.ds(cnp + off, SB)])
            pltpu.sync_copy(bbuf, agg.at[pl.ds(off, SB)])
            return carry2

        lax.fori_loop(0, NSB, blend_sub, 0)
        plsc.subcore_barrier()
        return carry

    lax.fori_loop(0, K, round_body, 0)


_appnp = functools.partial(
    pl.kernel,
    out_type=jax.ShapeDtypeStruct((2 * NP, HALF), jnp.float32),
    mesh=plsc.VectorSubcoreMesh(core_axis_name="c", subcore_axis_name="s"),
    compiler_params=pltpu.CompilerParams(use_tc_tiling_on_sc=False),
    scratch_types=[
        pltpu.VMEM((NCH2, C2), jnp.int32),
        pltpu.VMEM((NCH2, C2), jnp.int32),
        pltpu.VMEM((C2, HALF), jnp.float32),
        pltpu.VMEM((C2, HALF), jnp.float32),
        pltpu.VMEM((SB, HALF), jnp.float32),
        pltpu.VMEM((SEG, HALF), jnp.float32),
        pltpu.VMEM((SEG, 16), jnp.float32),
        pltpu.VMEM_SHARED((NP + 16, HALF), jnp.float32),
        pltpu.SemaphoreType.DMA,
        pltpu.SemaphoreType.DMA,
    ],
)(_appnp_body)


def _final_body(u_ref, deg2_ref, out_ref):
    deg = deg2_ref[0] + deg2_ref[1]
    srt = jnp.sqrt(deg[:N, 0:1])
    u64 = jnp.concatenate([u_ref[pl.ds(0, N), :], u_ref[pl.ds(NP, N), :]], axis=1)
    o = u64 * srt
    m = jnp.max(o, axis=1, keepdims=True)
    e = o - m
    lse = jnp.log(jnp.sum(jnp.exp(e), axis=1, keepdims=True))
    out_ref[...] = e - lse


_final = pl.pallas_call(
    _final_body,
    out_shape=jax.ShapeDtypeStruct((N, OUT_C), jnp.float32),
)


def kernel(x, edge_index, W0, b0, gamma, beta, W1, b1):
    edge_flat = edge_index.reshape(2 * E)
    deg2 = _deg_kernel(edge_flat)
    u0, g, coef = _mlp_prep(x, W0, b0, gamma, beta, W1, b1, deg2)
    u = _appnp(edge_flat, u0, coef, g)
    return _final(u, deg2)


# speedup vs baseline: 1.9289x; 1.1266x over previous
import functools

import jax
import jax.numpy as jnp
from jax import lax
from jax.experimental import pallas as pl
from jax.experimental.pallas import tpu as pltpu
from jax.experimental.pallas import tpu_sc as plsc

N = 10000
E = 320000
OUT_C = 64
HALF = 32
K = 10
ALPHA = 0.1
BN_EPS = 1e-5

NC = 2
NS = 16
NP = 10112
SEG = NP // NS
CH = 128

EPT = E // NS
C2 = 800
NCH2 = EPT // C2
SB = 158
NSB = SEG // SB

EPW = E // (NC * NS)
DC = 400
DNC = EPW // DC


def _mlp_prep_body(x_ref, w0_ref, b0_ref, gamma_ref, beta_ref, w1_ref, b1_ref,
                   deg2_ref, u0_ref, g_ref, coef_ref):
    x = x_ref[...]
    h = jnp.dot(x, w0_ref[...], preferred_element_type=jnp.float32)
    h = h + b0_ref[...][None, :]
    mu = jnp.mean(h, axis=0, keepdims=True)
    d = h - mu
    var = jnp.mean(d * d, axis=0, keepdims=True)
    h = d / jnp.sqrt(var + BN_EPS) * gamma_ref[...][None, :] + beta_ref[...][None, :]
    h = jnp.maximum(h, 0.0)
    out0 = (
        jnp.dot(h, w1_ref[...], preferred_element_type=jnp.float32)
        + b1_ref[...][None, :]
    )
    deg = deg2_ref[0] + deg2_ref[1]
    sinv = 1.0 / jnp.sqrt(deg)
    coef_ref[...] = (1.0 - ALPHA) * sinv * sinv
    s_n = sinv[:N, 0:1]
    u0 = out0 * s_n
    g = ALPHA * u0
    zpad = jnp.zeros((NP - N, HALF), jnp.float32)
    u0_ref[pl.ds(0, N), :] = u0[:, :HALF]
    u0_ref[pl.ds(N, NP - N), :] = zpad
    u0_ref[pl.ds(NP, N), :] = u0[:, HALF:]
    u0_ref[pl.ds(NP + N, NP - N), :] = zpad
    g_ref[pl.ds(0, N), :] = g[:, :HALF]
    g_ref[pl.ds(N, NP - N), :] = zpad
    g_ref[pl.ds(NP, N), :] = g[:, HALF:]
    g_ref[pl.ds(NP + N, NP - N), :] = zpad


_mlp_prep = pl.pallas_call(
    _mlp_prep_body,
    out_shape=[
        jax.ShapeDtypeStruct((2 * NP, HALF), jnp.float32),
        jax.ShapeDtypeStruct((2 * NP, HALF), jnp.float32),
        jax.ShapeDtypeStruct((NP, 16), jnp.float32),
    ],
)


def _deg_body(edge_ref, deg_out, idx0, idx1, ones, seg, degsp, sd0, sd1):
    c = lax.axis_index("c")
    s = lax.axis_index("s")
    w = s * NC + c
    ebase = w * EPW

    def fill_ones(i, carry):
        ones[i] = jnp.zeros((16,), jnp.float32) + 1.0
        return carry

    lax.fori_loop(0, DC, fill_ones, 0)

    def fill_seg0(i, carry):
        seg[i] = jnp.zeros((16,), jnp.float32)
        return carry

    lax.fori_loop(0, SEG, fill_seg0, 0)

    @pl.when(c == 0)
    def _fill_seg1():
        def fill_seg1(i, carry):
            seg[i] = jnp.zeros((16,), jnp.float32) + 1.0
            return carry

        lax.fori_loop(0, SEG, fill_seg1, 0)
    pltpu.sync_copy(seg, degsp.at[pl.ds(s * SEG, SEG)])
    plsc.subcore_barrier()

    idxs = (idx0, idx1)
    dsems = (sd0, sd1)

    def stage_issue(j, b):
        pltpu.async_copy(edge_ref.at[pl.ds(E + ebase + j * DC, DC)],
                         idxs[b].at[0], dsems[b])

    def stage_wait(b):
        pltpu.make_async_copy(edge_ref.at[pl.ds(0, DC)], idxs[b].at[0],
                              dsems[b]).wait()

    def dscat(b):
        pltpu.sync_copy(ones, degsp.at[idxs[b].at[0]], add=True)

    stage_issue(0, 0)

    def dpair(gp, carry):
        j0 = 2 * gp
        j1 = 2 * gp + 1
        stage_wait(0)
        stage_issue(j1, 1)
        dscat(0)
        stage_wait(1)

        @pl.when(j1 + 1 < DNC)
        def _():
            stage_issue(j1 + 1, 0)

        dscat(1)
        return carry

    lax.fori_loop(0, DNC // 2, dpair, 0)
    stage_wait(0)
    dscat(0)

    plsc.subcore_barrier()
    pltpu.sync_copy(degsp.at[pl.ds(s * SEG, SEG)], seg)
    pltpu.sync_copy(seg, deg_out.at[c, pl.ds(s * SEG, SEG)])


_deg_kernel = functools.partial(
    pl.kernel,
    out_type=jax.ShapeDtypeStruct((NC, NP, 16), jnp.float32),
    mesh=plsc.VectorSubcoreMesh(core_axis_name="c", subcore_axis_name="s"),
    compiler_params=pltpu.CompilerParams(use_tc_tiling_on_sc=False),
    scratch_types=[
        pltpu.VMEM((1, DC), jnp.int32),
        pltpu.VMEM((1, DC), jnp.int32),
        pltpu.VMEM((DC, 16), jnp.float32),
        pltpu.VMEM((SEG, 16), jnp.float32),
        pltpu.VMEM_SHARED((NP + 16, 16), jnp.float32),
        pltpu.SemaphoreType.DMA,
        pltpu.SemaphoreType.DMA,
    ],
)(_deg_body)


def _appnp_body(edge_ref, u0_ref, coef_ref, g_ref, u_ref,
                src_idx, dst_idx, gbuf0, gbuf1, bbuf, gvec,
                cvec, agg, sg0, sg1):
    c = lax.axis_index("c")
    t = lax.axis_index("s")
    ebase = t * EPT
    rbase = t * SEG
    cnp = c * NP

    def stage_row(j, carry):
        pltpu.sync_copy(edge_ref.at[pl.ds(ebase + j * C2, C2)], src_idx.at[j])
        pltpu.sync_copy(edge_ref.at[pl.ds(E + ebase + j * C2, C2)], dst_idx.at[j])
        return carry

    lax.fori_loop(0, NCH2, stage_row, 0)

    def add_off(j, carry):
        def add_lane(l, inner):
            v = src_idx[j, pl.ds(l * 16, 16)]
            src_idx[j, pl.ds(l * 16, 16)] = v + cnp
            return inner

        return lax.fori_loop(0, C2 // 16, add_lane, carry)

    lax.fori_loop(0, NCH2, add_off, 0)

    def init_sub(sb, carry):
        off = rbase + sb * SB
        pltpu.sync_copy(u0_ref.at[pl.ds(cnp + off, SB)], bbuf)
        pltpu.sync_copy(bbuf, u_ref.at[pl.ds(cnp + off, SB)])
        pltpu.sync_copy(bbuf, agg.at[pl.ds(off, SB)])
        return carry

    lax.fori_loop(0, NSB, init_sub, 0)
    plsc.subcore_barrier()

    bufs = (gbuf0, gbuf1)
    sems = (sg0, sg1)

    def gather_issue(j, b):
        pltpu.async_copy(u_ref.at[src_idx.at[j]], bufs[b], sems[b])

    def gather_wait(b):
        pltpu.make_async_copy(u_ref.at[pl.ds(0, C2)], bufs[b], sems[b]).wait()

    def round_body(k, carry):
        gather_issue(0, 0)

        def pair(gp, c2):
            j0 = 2 * gp
            j1 = 2 * gp + 1
            gather_wait(0)
            gather_issue(j1, 1)
            pltpu.sync_copy(bufs[0], agg.at[dst_idx.at[j0]], add=True)
            gather_wait(1)

            @pl.when(j1 + 1 < NCH2)
            def _():
                gather_issue(j1 + 1, 0)

            pltpu.sync_copy(bufs[1], agg.at[dst_idx.at[j1]], add=True)
            return c2

        lax.fori_loop(0, NCH2 // 2, pair, 0)
        gather_wait(0)
        pltpu.sync_copy(bufs[0], agg.at[dst_idx.at[NCH2 - 1]], add=True)
        plsc.subcore_barrier()

        def blend_sub(sb, carry2):
            off = rbase + sb * SB
            pltpu.sync_copy(coef_ref.at[pl.ds(off, SB)], cvec)
            pltpu.sync_copy(g_ref.at[pl.ds(cnp + off, SB)], gvec)
            pltpu.sync_copy(agg.at[pl.ds(off, SB)], bbuf)

            def blend_row(i, c3):
                ci = cvec[i][0]
                bbuf[i, pl.ds(0, 16)] = (
                    bbuf[i, pl.ds(0, 16)] * ci + gvec[i, pl.ds(0, 16)]
                )
                bbuf[i, pl.ds(16, 16)] = (
                    bbuf[i, pl.ds(16, 16)] * ci + gvec[i, pl.ds(16, 16)]
                )
                return c3

            lax.fori_loop(0, SB, blend_row, 0)
            pltpu.sync_copy(bbuf, u_ref.at[pl.ds(cnp + off, SB)])
            pltpu.sync_copy(bbuf, agg.at[pl.ds(off, SB)])
            return carry2

        lax.fori_loop(0, NSB, blend_sub, 0)
        plsc.subcore_barrier()
        return carry

    lax.fori_loop(0, K, round_body, 0)


_appnp = functools.partial(
    pl.kernel,
    out_type=jax.ShapeDtypeStruct((2 * NP, HALF), jnp.float32),
    mesh=plsc.VectorSubcoreMesh(core_axis_name="c", subcore_axis_name="s"),
    compiler_params=pltpu.CompilerParams(use_tc_tiling_on_sc=False),
    scratch_types=[
        pltpu.VMEM((NCH2, C2), jnp.int32),
        pltpu.VMEM((NCH2, C2), jnp.int32),
        pltpu.VMEM((C2, HALF), jnp.float32),
        pltpu.VMEM((C2, HALF), jnp.float32),
        pltpu.VMEM((SB, HALF), jnp.float32),
        pltpu.VMEM((SB, HALF), jnp.float32),
        pltpu.VMEM((SB, 16), jnp.float32),
        pltpu.VMEM_SHARED((NP + 16, HALF), jnp.float32),
        pltpu.SemaphoreType.DMA,
        pltpu.SemaphoreType.DMA,
    ],
)(_appnp_body)


def _final_body(u_ref, deg2_ref, out_ref):
    deg = deg2_ref[0] + deg2_ref[1]
    srt = jnp.sqrt(deg[:N, 0:1])
    u64 = jnp.concatenate([u_ref[pl.ds(0, N), :], u_ref[pl.ds(NP, N), :]], axis=1)
    o = u64 * srt
    m = jnp.max(o, axis=1, keepdims=True)
    e = o - m
    lse = jnp.log(jnp.sum(jnp.exp(e), axis=1, keepdims=True))
    out_ref[...] = e - lse


_final = pl.pallas_call(
    _final_body,
    out_shape=jax.ShapeDtypeStruct((N, OUT_C), jnp.float32),
)


def kernel(x, edge_index, W0, b0, gamma, beta, W1, b1):
    edge_flat = edge_index.reshape(2 * E)
    deg2 = _deg_kernel(edge_flat)
    u0, g, coef = _mlp_prep(x, W0, b0, gamma, beta, W1, b1, deg2)
    u = _appnp(edge_flat, u0, coef, g)
    return _final(u, deg2)
